# baseline (speedup 1.0000x reference)
import functools

import jax
import jax.numpy as jnp
from jax import lax
from jax.experimental import pallas as pl
from jax.experimental.pallas import tpu as pltpu
from jax.experimental.pallas import tpu_sc as plsc


@functools.lru_cache(maxsize=None)
def _make_gather(V, D, B):
    info = plsc.get_sparse_core_info()
    NC, NS = info.num_cores, info.num_subcores
    NW = NC * NS
    assert B % NW == 0
    b_per_w = B // NW
    C = 800
    while b_per_w % C:
        C //= 2
    n_chunks = b_per_w // C
    mesh = plsc.VectorSubcoreMesh(core_axis_name="c", subcore_axis_name="s")

    @functools.partial(
        pl.kernel,
        mesh=mesh,
        compiler_params=pltpu.CompilerParams(use_tc_tiling_on_sc=False),
        out_type=jax.ShapeDtypeStruct((B, D), jnp.float32),
        scratch_types=[
            pltpu.VMEM((C,), jnp.int32),
            pltpu.VMEM((C, D), jnp.float32),
            pltpu.SemaphoreType.DMA,
        ],
    )
    def gather_kernel(table_hbm, idx_hbm, out_hbm, idx_v, rows_v, sem):
        wid = lax.axis_index("s") * NC + lax.axis_index("c")
        base = wid * b_per_w
        for g in range(n_chunks):
            off = base + g * C
            pltpu.sync_copy(idx_hbm.at[pl.ds(off, C)], idx_v)
            pltpu.async_copy(table_hbm.at[idx_v], rows_v, sem).wait()
            pltpu.sync_copy(rows_v, out_hbm.at[pl.ds(off, C)])

    return gather_kernel


def kernel(inputs, word_embeddings):
    batch, seq = inputs.shape
    V, D = word_embeddings.shape
    idx = inputs.reshape(-1).astype(jnp.int32)
    out = _make_gather(V, D, batch * seq)(word_embeddings, idx)
    return out.reshape(batch, seq, D)


# speedup vs baseline: 1.0071x; 1.0071x over previous
import functools

import jax
import jax.numpy as jnp
from jax import lax
from jax.experimental import pallas as pl
from jax.experimental.pallas import tpu as pltpu
from jax.experimental.pallas import tpu_sc as plsc


def _chunk_size(b_per_w):
    C = 320
    while b_per_w % C:
        C //= 2
    return C


@functools.lru_cache(maxsize=None)
def _make_gather(V, D, B):
    info = plsc.get_sparse_core_info()
    NC, NS = info.num_cores, info.num_subcores
    NW = NC * NS
    assert B % NW == 0
    b_per_w = B // NW
    C = _chunk_size(b_per_w)
    n_chunks = b_per_w // C
    NBUF = min(5, n_chunks)
    AHEAD = max(NBUF - 3, 0)
    mesh = plsc.VectorSubcoreMesh(core_axis_name="c", subcore_axis_name="s")

    @functools.partial(
        pl.kernel,
        mesh=mesh,
        compiler_params=pltpu.CompilerParams(use_tc_tiling_on_sc=False),
        out_type=jax.ShapeDtypeStruct((B, D), jnp.float32),
        scratch_types=[
            pltpu.VMEM((n_chunks, C), jnp.int32),
            pltpu.VMEM((NBUF, C, D), jnp.float32),
        ]
        + [pltpu.SemaphoreType.DMA] * (2 * NBUF),
    )
    def gather_kernel(table_hbm, idx_hbm, out_hbm, idx_v, rows_v, *sems):
        gsems, wsems = sems[:NBUF], sems[NBUF:]
        wid = lax.axis_index("s") * NC + lax.axis_index("c")
        base = wid * b_per_w
        pltpu.sync_copy(idx_hbm.at[pl.ds(wid * n_chunks, n_chunks)], idx_v)

        def start_gather(g):
            b = g % NBUF
            return pltpu.async_copy(
                table_hbm.at[idx_v.at[g]], rows_v.at[b], gsems[b]
            )

        def start_writeout(g):
            b = g % NBUF
            return pltpu.async_copy(
                rows_v.at[b], out_hbm.at[pl.ds(base + g * C, C)], wsems[b]
            )

        ghandles, whandles = {}, {}
        for g in range(min(AHEAD + 1, n_chunks)):
            ghandles[g] = start_gather(g)
        for g in range(n_chunks):
            ghandles.pop(g).wait()
            whandles[g] = start_writeout(g)
            p = g + AHEAD + 1
            if p < n_chunks:
                old = p - NBUF
                if old >= 0:
                    whandles.pop(old).wait()
                ghandles[p] = start_gather(p)
        for g in sorted(whandles):
            whandles.pop(g).wait()

    return gather_kernel


def kernel(inputs, word_embeddings):
    batch, seq = inputs.shape
    V, D = word_embeddings.shape
    B = batch * seq
    info = plsc.get_sparse_core_info()
    C = _chunk_size(B // (info.num_cores * info.num_subcores))
    idx = inputs.reshape(B // C, C).astype(jnp.int32)
    out = _make_gather(V, D, B)(word_embeddings, idx)
    return out.reshape(batch, seq, D)
